# baseline (speedup 1.0000x reference)
import jax
import jax.numpy as jnp
from jax.experimental import pallas as pl
from jax.experimental.pallas import tpu as pltpu

NTYPE = 3
N = 2048
NFEAT = 128
NHID = 64
NCLS = 32 + NTYPE - 1
ATT = 50
GAMMA = 0.1
BR = 256
NB = N // BR


def _leaky(x):
    return jnp.maximum(x, 0.2 * x)


def _body(a00, a01, a02, a10, a11, a12, a20, a21, a22,
          x0, x1, x2, wg, a2s, a1c, wat1, bat1, aat1_a, aat1_b, w2,
          b2, wat2, bat2, aat2_a, aat2_b,
          o0, o1, o2,
          he0, he1, he2, hb0, hb1, hb2, f2s, ys0, ys1, ys2):
    i = pl.program_id(0)
    bf = jnp.bfloat16
    adj = ((a00, a01, a02), (a10, a11, a12), (a20, a21, a22))
    xs = (x0, x1, x2)
    hes = (he0, he1, he2)
    hbs = (hb0, hb1, hb2)
    yss = (ys0, ys1, ys2)
    outs = (o0, o1, o2)

    @pl.when(i == 0)
    def _prep():
        ones = jnp.ones((N, 1), jnp.float32)
        for t in range(NTYPE):
            h = jnp.dot(xs[t][...], wg[t], preferred_element_type=jnp.float32)
            hes[t][:, :NHID] = h
            hes[t][:, NHID : NHID + 1] = ones
            hbs[t][:, :NHID] = h.astype(bf)
            hbs[t][:, NHID : NHID + 1] = ones.astype(bf)
            col = jnp.dot(h, a2s[:, t : t + 1],
                          preferred_element_type=jnp.float32)
            f2s[t : t + 1, :] = col.T.astype(bf)

    @pl.when(i < NB)
    def _layer1():
        f2 = f2s[...]
        for t1 in range(NTYPE):
            heblk = hes[t1][pl.ds(i * BR, BR), :]
            f1all = jnp.dot(heblk, a1c[...],
                            preferred_element_type=jnp.float32)
            f1bf = f1all.astype(bf)
            cols = []
            for t2 in range(NTYPE):
                A = adj[t1][t2][...]
                abf = A.astype(bf)
                hfull = hbs[t2][...]
                e = _leaky(f1bf[:, t2 : t2 + 1] + f2[t2 : t2 + 1, :])
                p = jnp.where(abf > 0, jnp.exp2(e), bf(0.0))
                ph = jnp.dot(p, hfull, preferred_element_type=jnp.float32)
                ah = jnp.dot(abf, hfull, preferred_element_type=jnp.float32)
                s = ph[:, NHID : NHID + 1]
                sinv = GAMMA / jnp.maximum(s, 1e-30)
                cols.append(ph[:, :NHID] * sinv + ah[:, :NHID] * (1.0 - GAMMA))
            xatt = [jnp.tanh(jnp.dot(cols[t2], wat1[t1],
                                     preferred_element_type=jnp.float32)
                             + bat1[t1]) for t2 in range(NTYPE)]
            e0 = jnp.dot(xatt[t1], aat1_a[:, t1 : t1 + 1],
                         preferred_element_type=jnp.float32)
            es = [_leaky(e0 + jnp.dot(xatt[t2], aat1_b[:, t1 : t1 + 1],
                                      preferred_element_type=jnp.float32))
                  for t2 in range(NTYPE)]
            m = jnp.maximum(jnp.maximum(es[0], es[1]), es[2])
            ws = [jnp.exp(es[t2] - m) for t2 in range(NTYPE)]
            denom = ws[0] + ws[1] + ws[2]
            out = (cols[0] * ws[0] + cols[1] * ws[1] + cols[2] * ws[2]) / denom
            out = jnp.maximum(out, 0.0)
            yss[t1][pl.ds(i * BR, BR), :] = jnp.dot(
                out, w2[...], preferred_element_type=jnp.float32)

    @pl.when(i >= NB)
    def _layer2():
        yfull = [yss[t][...] for t in range(NTYPE)]
        brow = b2[...]
        for t1 in range(NTYPE):
            cols = [jnp.dot(adj[t1][t2][...], yfull[t2],
                            preferred_element_type=jnp.float32) + brow
                    for t2 in range(NTYPE)]
            xatt = [jnp.tanh(jnp.dot(cols[t2], wat2[t1],
                                     preferred_element_type=jnp.float32)
                             + bat2[t1]) for t2 in range(NTYPE)]
            e0 = jnp.dot(xatt[t1], aat2_a[:, t1 : t1 + 1],
                         preferred_element_type=jnp.float32)
            es = [_leaky(e0 + jnp.dot(xatt[t2], aat2_b[:, t1 : t1 + 1],
                                      preferred_element_type=jnp.float32))
                  for t2 in range(NTYPE)]
            m = jnp.maximum(jnp.maximum(es[0], es[1]), es[2])
            ws = [jnp.exp(es[t2] - m) for t2 in range(NTYPE)]
            denom = ws[0] + ws[1] + ws[2]
            out = (cols[0] * ws[0] + cols[1] * ws[1] + cols[2] * ws[2]) / denom
            mm = jnp.max(out, axis=1, keepdims=True)
            lse = jnp.log(jnp.sum(jnp.exp(out - mm), axis=1,
                                  keepdims=True)) + mm
            outs[t1][...] = out - lse


def kernel(x_list, adj_list, Wgc1, a1, a2, W2, b2, Wat1, bat1, aat1,
           Wat2, bat2, aat2):
    LOG2E = 1.4426950408889634
    wg = jnp.stack(Wgc1)
    a1c = jnp.concatenate(
        [jnp.concatenate(a1, axis=1) * LOG2E,
         jnp.zeros((1, NTYPE), jnp.float32)], axis=0)
    a2s = jnp.concatenate(a2, axis=1) * LOG2E
    wat1 = jnp.stack(Wat1)
    bat1r = jnp.stack(bat1)[:, None, :]
    aat1_a = jnp.concatenate([v[:ATT] for v in aat1], axis=1)
    aat1_b = jnp.concatenate([v[ATT:] for v in aat1], axis=1)
    wat2 = jnp.stack(Wat2)
    bat2r = jnp.stack(bat2)[:, None, :]
    aat2_a = jnp.concatenate([v[:ATT] for v in aat2], axis=1)
    aat2_b = jnp.concatenate([v[ATT:] for v in aat2], axis=1)
    b2row = b2[None, :]

    adj_spec = pl.BlockSpec((BR, N), lambda i: (jax.lax.rem(i, NB), 0))
    out_spec = pl.BlockSpec(
        (BR, NCLS), lambda i: (jnp.maximum(i - NB, 0), 0))
    small = lambda shp: pl.BlockSpec(shp, lambda i: tuple(0 for _ in shp))
    o0, o1, o2 = pl.pallas_call(
        _body,
        grid=(2 * NB,),
        in_specs=[adj_spec] * 9 + [
            small((N, NFEAT)), small((N, NFEAT)), small((N, NFEAT)),
            small((NTYPE, NFEAT, NHID)), small((NHID, NTYPE)),
            small((NHID + 1, NTYPE)), small((NTYPE, NHID, ATT)),
            small((NTYPE, 1, ATT)), small((ATT, NTYPE)), small((ATT, NTYPE)),
            small((NHID, NCLS)), small((1, NCLS)), small((NTYPE, NCLS, ATT)),
            small((NTYPE, 1, ATT)), small((ATT, NTYPE)), small((ATT, NTYPE)),
        ],
        out_specs=[out_spec] * 3,
        out_shape=[jax.ShapeDtypeStruct((N, NCLS), jnp.float32)] * 3,
        scratch_shapes=[
            pltpu.VMEM((N, NHID + 1), jnp.float32),
            pltpu.VMEM((N, NHID + 1), jnp.float32),
            pltpu.VMEM((N, NHID + 1), jnp.float32),
            pltpu.VMEM((N, NHID + 1), jnp.bfloat16),
            pltpu.VMEM((N, NHID + 1), jnp.bfloat16),
            pltpu.VMEM((N, NHID + 1), jnp.bfloat16),
            pltpu.VMEM((NTYPE, N), jnp.bfloat16),
            pltpu.VMEM((N, NCLS), jnp.float32),
            pltpu.VMEM((N, NCLS), jnp.float32),
            pltpu.VMEM((N, NCLS), jnp.float32),
        ],
        compiler_params=pltpu.CompilerParams(
            dimension_semantics=("arbitrary",)),
    )(adj_list[0][0], adj_list[0][1], adj_list[0][2],
      adj_list[1][0], adj_list[1][1], adj_list[1][2],
      adj_list[2][0], adj_list[2][1], adj_list[2][2],
      x_list[0], x_list[1], x_list[2], wg, a2s, a1c,
      wat1, bat1r, aat1_a, aat1_b, W2,
      b2row, wat2, bat2r, aat2_a, aat2_b)
    return (o0, o1, o2)


# speedup vs baseline: 1.0125x; 1.0125x over previous
import jax
import jax.numpy as jnp
from jax.experimental import pallas as pl
from jax.experimental.pallas import tpu as pltpu

NTYPE = 3
N = 2048
NFEAT = 128
NHID = 64
NCLS = 32 + NTYPE - 1
ATT = 50
GAMMA = 0.1
BR = 256
NB = N // BR


def _leaky(x):
    return jnp.maximum(x, 0.2 * x)


def _body(a00, a01, a02, a10, a11, a12, a20, a21, a22,
          x0, x1, x2, wg, a2s, a1c, wat1, bat1, aat1_a, aat1_b, w2,
          b2, wat2, bat2, aat2_a, aat2_b,
          o0, o1, o2,
          he0, he1, he2, hb0, hb1, hb2, f2s, ys0, ys1, ys2):
    i = pl.program_id(0)
    bf = jnp.bfloat16
    adj = ((a00, a01, a02), (a10, a11, a12), (a20, a21, a22))
    xs = (x0, x1, x2)
    hes = (he0, he1, he2)
    hbs = (hb0, hb1, hb2)
    yss = (ys0, ys1, ys2)
    outs = (o0, o1, o2)

    @pl.when(i == 0)
    def _prep():
        ones = jnp.ones((N, 1), jnp.float32)
        for t in range(NTYPE):
            h = jnp.dot(xs[t][...], wg[t], preferred_element_type=jnp.float32)
            hes[t][:, :NHID] = h
            hes[t][:, NHID : NHID + 1] = ones
            hbs[t][:, :NHID] = h.astype(bf)
            hbs[t][:, NHID : NHID + 1] = ones.astype(bf)
            col = jnp.dot(h, a2s[:, t : t + 1],
                          preferred_element_type=jnp.float32)
            f2s[t : t + 1, :] = col.T.astype(bf)

    @pl.when(i < NB)
    def _layer1():
        f2 = f2s[...]
        for t1 in range(NTYPE):
            heblk = hes[t1][pl.ds(i * BR, BR), :]
            f1all = jnp.dot(heblk, a1c[...],
                            preferred_element_type=jnp.float32)
            f1bf = f1all.astype(bf)
            cols = []
            for t2 in range(NTYPE):
                A = adj[t1][t2][...]
                abf = A.astype(bf)
                hfull = hbs[t2][...]
                e = _leaky(f1bf[:, t2 : t2 + 1] + f2[t2 : t2 + 1, :])
                p = jnp.where(abf > 0, jnp.exp2(e), bf(0.0))
                ph = jnp.dot(p, hfull, preferred_element_type=jnp.float32)
                ah = jnp.dot(abf, hfull, preferred_element_type=jnp.float32)
                s = ph[:, NHID : NHID + 1]
                sinv = GAMMA / jnp.maximum(s, 1e-30)
                cols.append(ph[:, :NHID] * sinv + ah[:, :NHID] * (1.0 - GAMMA))
            xatt = [jnp.tanh(jnp.dot(cols[t2], wat1[t1],
                                     preferred_element_type=jnp.float32)
                             + bat1[t1]) for t2 in range(NTYPE)]
            e0 = jnp.dot(xatt[t1], aat1_a[:, t1 : t1 + 1],
                         preferred_element_type=jnp.float32)
            es = [_leaky(e0 + jnp.dot(xatt[t2], aat1_b[:, t1 : t1 + 1],
                                      preferred_element_type=jnp.float32))
                  for t2 in range(NTYPE)]
            m = jnp.maximum(jnp.maximum(es[0], es[1]), es[2])
            ws = [jnp.exp(es[t2] - m) for t2 in range(NTYPE)]
            denom = ws[0] + ws[1] + ws[2]
            out = (cols[0] * ws[0] + cols[1] * ws[1] + cols[2] * ws[2]) / denom
            out = jnp.maximum(out, 0.0)
            yss[t1][pl.ds(i * BR, BR), :] = jnp.dot(
                out, w2[...], preferred_element_type=jnp.float32)

    @pl.when(i >= NB)
    def _layer2():
        yfull = [yss[t][...].astype(bf) for t in range(NTYPE)]
        brow = b2[...]
        for t1 in range(NTYPE):
            cols = [jnp.dot(adj[t1][t2][...].astype(bf), yfull[t2],
                            preferred_element_type=jnp.float32) + brow
                    for t2 in range(NTYPE)]
            xatt = [jnp.tanh(jnp.dot(cols[t2], wat2[t1],
                                     preferred_element_type=jnp.float32)
                             + bat2[t1]) for t2 in range(NTYPE)]
            e0 = jnp.dot(xatt[t1], aat2_a[:, t1 : t1 + 1],
                         preferred_element_type=jnp.float32)
            es = [_leaky(e0 + jnp.dot(xatt[t2], aat2_b[:, t1 : t1 + 1],
                                      preferred_element_type=jnp.float32))
                  for t2 in range(NTYPE)]
            m = jnp.maximum(jnp.maximum(es[0], es[1]), es[2])
            ws = [jnp.exp(es[t2] - m) for t2 in range(NTYPE)]
            denom = ws[0] + ws[1] + ws[2]
            out = (cols[0] * ws[0] + cols[1] * ws[1] + cols[2] * ws[2]) / denom
            mm = jnp.max(out, axis=1, keepdims=True)
            lse = jnp.log(jnp.sum(jnp.exp(out - mm), axis=1,
                                  keepdims=True)) + mm
            outs[t1][...] = out - lse


def kernel(x_list, adj_list, Wgc1, a1, a2, W2, b2, Wat1, bat1, aat1,
           Wat2, bat2, aat2):
    LOG2E = 1.4426950408889634
    wg = jnp.stack(Wgc1)
    a1c = jnp.concatenate(
        [jnp.concatenate(a1, axis=1) * LOG2E,
         jnp.zeros((1, NTYPE), jnp.float32)], axis=0)
    a2s = jnp.concatenate(a2, axis=1) * LOG2E
    wat1 = jnp.stack(Wat1)
    bat1r = jnp.stack(bat1)[:, None, :]
    aat1_a = jnp.concatenate([v[:ATT] for v in aat1], axis=1)
    aat1_b = jnp.concatenate([v[ATT:] for v in aat1], axis=1)
    wat2 = jnp.stack(Wat2)
    bat2r = jnp.stack(bat2)[:, None, :]
    aat2_a = jnp.concatenate([v[:ATT] for v in aat2], axis=1)
    aat2_b = jnp.concatenate([v[ATT:] for v in aat2], axis=1)
    b2row = b2[None, :]

    adj_spec = pl.BlockSpec((BR, N), lambda i: (jax.lax.rem(i, NB), 0))
    out_spec = pl.BlockSpec(
        (BR, NCLS), lambda i: (jnp.maximum(i - NB, 0), 0))
    small = lambda shp: pl.BlockSpec(shp, lambda i: tuple(0 for _ in shp))
    o0, o1, o2 = pl.pallas_call(
        _body,
        grid=(2 * NB,),
        in_specs=[adj_spec] * 9 + [
            small((N, NFEAT)), small((N, NFEAT)), small((N, NFEAT)),
            small((NTYPE, NFEAT, NHID)), small((NHID, NTYPE)),
            small((NHID + 1, NTYPE)), small((NTYPE, NHID, ATT)),
            small((NTYPE, 1, ATT)), small((ATT, NTYPE)), small((ATT, NTYPE)),
            small((NHID, NCLS)), small((1, NCLS)), small((NTYPE, NCLS, ATT)),
            small((NTYPE, 1, ATT)), small((ATT, NTYPE)), small((ATT, NTYPE)),
        ],
        out_specs=[out_spec] * 3,
        out_shape=[jax.ShapeDtypeStruct((N, NCLS), jnp.float32)] * 3,
        scratch_shapes=[
            pltpu.VMEM((N, NHID + 1), jnp.float32),
            pltpu.VMEM((N, NHID + 1), jnp.float32),
            pltpu.VMEM((N, NHID + 1), jnp.float32),
            pltpu.VMEM((N, NHID + 1), jnp.bfloat16),
            pltpu.VMEM((N, NHID + 1), jnp.bfloat16),
            pltpu.VMEM((N, NHID + 1), jnp.bfloat16),
            pltpu.VMEM((NTYPE, N), jnp.bfloat16),
            pltpu.VMEM((N, NCLS), jnp.float32),
            pltpu.VMEM((N, NCLS), jnp.float32),
            pltpu.VMEM((N, NCLS), jnp.float32),
        ],
        compiler_params=pltpu.CompilerParams(
            dimension_semantics=("arbitrary",)),
    )(adj_list[0][0], adj_list[0][1], adj_list[0][2],
      adj_list[1][0], adj_list[1][1], adj_list[1][2],
      adj_list[2][0], adj_list[2][1], adj_list[2][2],
      x_list[0], x_list[1], x_list[2], wg, a2s, a1c,
      wat1, bat1r, aat1_a, aat1_b, W2,
      b2row, wat2, bat2r, aat2_a, aat2_b)
    return (o0, o1, o2)
